# baseline (speedup 1.0000x reference)
import functools

import jax
import jax.numpy as jnp
from jax import lax
from jax.experimental import pallas as pl
from jax.experimental.pallas import tpu as pltpu
from jax.experimental.pallas import tpu_sc as plsc

NUM_AUTHORS = 50000
SPAN = 50000
DIM = 128
NW = 32
NSUB = 16

SIZES = (224,) * 7
OFFS = tuple(sum(SIZES[:j]) for j in range(len(SIZES)))
PW = sum(SIZES)
SLOT = (0, 1, 3, 2, 0, 3, 1)
SLOT_SIZE = (224, 224, 224, 224)
SPMEM_SLOTS = ()
NCH = len(SIZES)
_prev_user = {j: max((k for k in range(j) if SLOT[k] == SLOT[j]), default=None)
              for j in range(NCH)}

_mesh = plsc.VectorSubcoreMesh(core_axis_name="c", subcore_axis_name="s")


def _make_slice_copy(srow):
    @functools.partial(
        pl.kernel,
        out_type=jax.ShapeDtypeStruct((SPAN, DIM), jnp.float32),
        mesh=_mesh,
        scratch_types=(
            [pltpu.VMEM((SLOT_SIZE[s], DIM), jnp.float32)
             for s in range(len(SLOT_SIZE)) if s not in SPMEM_SLOTS]
            + [pltpu.VMEM_SHARED((NSUB, SLOT_SIZE[s], DIM), jnp.float32)
               for s in SPMEM_SLOTS]
            + [pltpu.SemaphoreType.DMA for _ in range(2 * len(SLOT_SIZE))]
        ),
        name=f"sc_slice_copy_{srow}",
    )
    def _sc_slice_copy(table_hbm, out_hbm, *scratch):
        nslots = len(SLOT_SIZE)
        tile_ids = [s for s in range(nslots) if s not in SPMEM_SLOTS]
        refs = {}
        for i, s in enumerate(tile_ids):
            refs[s] = scratch[i]
        for i, s in enumerate(SPMEM_SLOTS):
            refs[s] = scratch[len(tile_ids) + i]
        sems = scratch[len(tile_ids) + len(SPMEM_SLOTS):]
        wid = lax.axis_index("s") * 2 + lax.axis_index("c")
        sid = lax.axis_index("s")
        base = pl.multiple_of(jnp.minimum(wid * PW, SPAN - PW), 8)

        def buf(j):
            s = SLOT[j]
            return refs[s].at[sid] if s in SPMEM_SLOTS else refs[s]

        def start_read(j):
            return pltpu.async_copy(
                table_hbm.at[pl.ds(
                    pl.multiple_of(srow + base + OFFS[j], 8), SIZES[j])],
                buf(j), sems[2 * SLOT[j]])

        def start_write(j):
            return pltpu.async_copy(
                buf(j),
                out_hbm.at[pl.ds(pl.multiple_of(base + OFFS[j], 8), SIZES[j])],
                sems[2 * SLOT[j] + 1])

        reads = {}
        writes = {}
        done = set()
        for j in range(NCH + 1):
            if j < NCH:
                prev = _prev_user[j]
                if prev is not None:
                    writes[prev].wait()
                    done.add(prev)
                reads[j] = start_read(j)
            if j >= 1:
                reads[j - 1].wait()
                writes[j - 1] = start_write(j - 1)
        for j in range(NCH):
            if j not in done:
                writes[j].wait()

    return _sc_slice_copy


_branches = tuple(_make_slice_copy(s) for s in (0, NUM_AUTHORS))


def kernel(node_type, embedding_weight):
    idx = jnp.asarray(node_type, dtype=jnp.int32)
    return lax.switch(idx, _branches, embedding_weight)


# speedup vs baseline: 1.0030x; 1.0030x over previous
import functools

import jax
import jax.numpy as jnp
from jax import lax
from jax.experimental import pallas as pl
from jax.experimental.pallas import tpu as pltpu
from jax.experimental.pallas import tpu_sc as plsc

NUM_AUTHORS = 50000
SPAN = 50000
DIM = 128
NW = 32
NSUB = 16

SIZES = (224,) * 7
OFFS = tuple(sum(SIZES[:j]) for j in range(len(SIZES)))
PW = sum(SIZES)
SLOT = (0, 1, 2, 3, 0, 2, 1)
SLOT_SIZE = (224, 224, 224, 224)
SPMEM_SLOTS = (2, 3)
NCH = len(SIZES)
_prev_user = {j: max((k for k in range(j) if SLOT[k] == SLOT[j]), default=None)
              for j in range(NCH)}

_mesh = plsc.VectorSubcoreMesh(core_axis_name="c", subcore_axis_name="s")


def _make_slice_copy(srow):
    @functools.partial(
        pl.kernel,
        out_type=jax.ShapeDtypeStruct((SPAN, DIM), jnp.float32),
        mesh=_mesh,
        scratch_types=(
            [pltpu.VMEM((SLOT_SIZE[s], DIM), jnp.float32)
             for s in range(len(SLOT_SIZE)) if s not in SPMEM_SLOTS]
            + [pltpu.VMEM_SHARED((NSUB, SLOT_SIZE[s], DIM), jnp.float32)
               for s in SPMEM_SLOTS]
            + [pltpu.SemaphoreType.DMA for _ in range(2 * len(SLOT_SIZE))]
        ),
        name=f"sc_slice_copy_{srow}",
    )
    def _sc_slice_copy(table_hbm, out_hbm, *scratch):
        nslots = len(SLOT_SIZE)
        tile_ids = [s for s in range(nslots) if s not in SPMEM_SLOTS]
        refs = {}
        for i, s in enumerate(tile_ids):
            refs[s] = scratch[i]
        for i, s in enumerate(SPMEM_SLOTS):
            refs[s] = scratch[len(tile_ids) + i]
        sems = scratch[len(tile_ids) + len(SPMEM_SLOTS):]
        wid = lax.axis_index("s") * 2 + lax.axis_index("c")
        sid = lax.axis_index("s")
        base = pl.multiple_of(jnp.minimum(wid * PW, SPAN - PW), 8)

        def buf(j):
            s = SLOT[j]
            return refs[s].at[sid] if s in SPMEM_SLOTS else refs[s]

        def start_read(j):
            return pltpu.async_copy(
                table_hbm.at[pl.ds(
                    pl.multiple_of(srow + base + OFFS[j], 8), SIZES[j])],
                buf(j), sems[2 * SLOT[j]])

        def start_write(j):
            return pltpu.async_copy(
                buf(j),
                out_hbm.at[pl.ds(pl.multiple_of(base + OFFS[j], 8), SIZES[j])],
                sems[2 * SLOT[j] + 1])

        reads = {}
        writes = {}
        done = set()
        for j in range(NCH + 1):
            if j < NCH:
                prev = _prev_user[j]
                if prev is not None:
                    writes[prev].wait()
                    done.add(prev)
                reads[j] = start_read(j)
            if j >= 1:
                reads[j - 1].wait()
                writes[j - 1] = start_write(j - 1)
        for j in range(NCH):
            if j not in done:
                writes[j].wait()

    return _sc_slice_copy


_branches = tuple(_make_slice_copy(s) for s in (0, NUM_AUTHORS))


def kernel(node_type, embedding_weight):
    idx = jnp.asarray(node_type, dtype=jnp.int32)
    return lax.switch(idx, _branches, embedding_weight)


# speedup vs baseline: 1.0483x; 1.0452x over previous
import functools

import jax
import jax.numpy as jnp
from jax import lax
from jax.experimental import pallas as pl
from jax.experimental.pallas import tpu as pltpu
from jax.experimental.pallas import tpu_sc as plsc

NUM_AUTHORS = 50000
SPAN = 50000
DIM = 128
NW = 32
NSUB = 16

SIZES = (224,) * 7
OFFS = tuple(sum(SIZES[:j]) for j in range(len(SIZES)))
PW = sum(SIZES)
SLOT = (0, 1, 3, 2, 0, 3, 1)
SLOT_SIZE = (224, 224, 224, 224)
SPMEM_SLOTS = (3,)
NCH = len(SIZES)
_prev_user = {j: max((k for k in range(j) if SLOT[k] == SLOT[j]), default=None)
              for j in range(NCH)}

_mesh = plsc.VectorSubcoreMesh(core_axis_name="c", subcore_axis_name="s")


def _make_slice_copy(srow):
    @functools.partial(
        pl.kernel,
        out_type=jax.ShapeDtypeStruct((SPAN, DIM), jnp.float32),
        mesh=_mesh,
        scratch_types=(
            [pltpu.VMEM((SLOT_SIZE[s], DIM), jnp.float32)
             for s in range(len(SLOT_SIZE)) if s not in SPMEM_SLOTS]
            + [pltpu.VMEM_SHARED((NSUB, SLOT_SIZE[s], DIM), jnp.float32)
               for s in SPMEM_SLOTS]
            + [pltpu.SemaphoreType.DMA for _ in range(2 * len(SLOT_SIZE))]
        ),
        name=f"sc_slice_copy_{srow}",
    )
    def _sc_slice_copy(table_hbm, out_hbm, *scratch):
        nslots = len(SLOT_SIZE)
        tile_ids = [s for s in range(nslots) if s not in SPMEM_SLOTS]
        refs = {}
        for i, s in enumerate(tile_ids):
            refs[s] = scratch[i]
        for i, s in enumerate(SPMEM_SLOTS):
            refs[s] = scratch[len(tile_ids) + i]
        sems = scratch[len(tile_ids) + len(SPMEM_SLOTS):]
        wid = lax.axis_index("s") * 2 + lax.axis_index("c")
        sid = lax.axis_index("s")
        base = pl.multiple_of(jnp.minimum(wid * PW, SPAN - PW), 8)

        def buf(j):
            s = SLOT[j]
            return refs[s].at[sid] if s in SPMEM_SLOTS else refs[s]

        def start_read(j):
            return pltpu.async_copy(
                table_hbm.at[pl.ds(
                    pl.multiple_of(srow + base + OFFS[j], 8), SIZES[j])],
                buf(j), sems[2 * SLOT[j]])

        def start_write(j):
            return pltpu.async_copy(
                buf(j),
                out_hbm.at[pl.ds(pl.multiple_of(base + OFFS[j], 8), SIZES[j])],
                sems[2 * SLOT[j] + 1])

        reads = {}
        writes = {}
        done = set()
        for j in range(NCH + 1):
            if j < NCH:
                prev = _prev_user[j]
                if prev is not None:
                    writes[prev].wait()
                    done.add(prev)
                reads[j] = start_read(j)
            if j >= 1:
                reads[j - 1].wait()
                writes[j - 1] = start_write(j - 1)
        for j in range(NCH):
            if j not in done:
                writes[j].wait()

    return _sc_slice_copy


_branches = tuple(_make_slice_copy(s) for s in (0, NUM_AUTHORS))


def kernel(node_type, embedding_weight):
    idx = jnp.asarray(node_type, dtype=jnp.int32)
    return lax.switch(idx, _branches, embedding_weight)
